# baseline (speedup 1.0000x reference)
import jax
import jax.numpy as jnp
from jax import lax
from jax.experimental import pallas as pl
from jax.experimental.pallas import tpu as pltpu
from jax.experimental.pallas import tpu_sc as plsc

N_STIMULI = 20
N_DIM = 3
BETA = 10.0
GAMMA = 0.001
BATCH = 16384

NUM_CORES = 2
NUM_SUBCORES = 16
LANES = 16
NUM_WORKERS = NUM_CORES * NUM_SUBCORES
B_PER_W = BATCH // NUM_WORKERS
STEPS = B_PER_W // LANES
TABLE_WORDS = (N_STIMULI + 1) * N_DIM
NV = N_STIMULI + 1
NPAIR = NV * NV
NPAIR_PAD = 448
NCHUNK = NPAIR_PAD // LANES


def _sqrt16(x):
    i = plsc.bitcast(x, jnp.int32)
    i = jnp.int32(0x5F3759DF) - lax.shift_right_arithmetic(i, 1)
    y = plsc.bitcast(i, jnp.float32)
    xh = x * jnp.float32(0.5)
    for _ in range(3):
        y = y * (jnp.float32(1.5) - xh * y * y)
    return x * y


def _pair_similarity(emb_v, p):
    q = p // jnp.int32(NV)
    r = p - q * jnp.int32(NV)
    dsq = jnp.full((LANES,), 0.0, jnp.float32)
    for d in range(N_DIM):
        dd = jnp.full((LANES,), d * NV, jnp.int32)
        t = plsc.load_gather(emb_v, [dd + q]) - plsc.load_gather(emb_v, [dd + r])
        dsq = dsq + t * t
    return jnp.exp(jnp.float32(-BETA) * _sqrt16(dsq)) + jnp.float32(GAMMA)


def _sc_body(widx_hbm, emb_hbm, out_hbm,
             w_v, emb_v, stab_v, sbuf_v, po_v, spmem, sem, sem2):
    sid = lax.axis_index("s")
    wid = sid * NUM_CORES + lax.axis_index("c")
    base = wid * B_PER_W

    ce = pltpu.async_copy(emb_hbm, emb_v, sem2)
    cw = pltpu.async_copy(widx_hbm.at[pl.ds(base, B_PER_W)], w_v, sem)
    ce.wait()

    lanes = lax.iota(jnp.int32, LANES)

    p0 = jnp.minimum(sid * LANES + lanes, jnp.int32(NPAIR - 1))
    sbuf_v[...] = _pair_similarity(emb_v, p0)
    pltpu.sync_copy(sbuf_v, spmem.at[pl.ds(sid * LANES, LANES)])

    @pl.when(sid + 16 < NCHUNK)
    def _():
        p1 = jnp.minimum((sid + 16) * LANES + lanes, jnp.int32(NPAIR - 1))
        sbuf_v[...] = _pair_similarity(emb_v, p1)
        pltpu.sync_copy(sbuf_v, spmem.at[pl.ds((sid + 16) * LANES, LANES)])

    plsc.subcore_barrier()
    pltpu.sync_copy(spmem, stab_v)
    cw.wait()

    mask = jnp.full((LANES,), 31, jnp.int32)

    outs = []
    for step in range(STEPS):
        off = step * LANES
        t = off // 128
        c = off % 128
        w = w_v[pl.ds(off, LANES)]
        q21 = (w & mask) * jnp.int32(NV)
        r1 = lax.shift_right_logical(w, 5) & mask
        r2 = lax.shift_right_logical(w, 10)
        s1 = plsc.load_gather(stab_v, [q21 + r1])
        s2 = plsc.load_gather(stab_v, [q21 + r2])
        inv = jnp.float32(1.0) / (s1 + s2)
        po_v[t, 0, pl.ds(c, LANES)] = s1 * inv
        po_v[t, 1, pl.ds(c, LANES)] = s2 * inv
        if c + LANES == 128:
            outs.append(pltpu.async_copy(
                po_v.at[pl.ds(t, 1)], out_hbm.at[pl.ds(wid * 4 + t, 1)], sem))
    for o in outs:
        o.wait()


@jax.jit
def kernel(stimulus_set, embedding):
    mesh = plsc.VectorSubcoreMesh(
        core_axis_name="c", subcore_axis_name="s",
        num_cores=NUM_CORES, num_subcores=NUM_SUBCORES,
    )
    out = pl.kernel(
        _sc_body,
        out_type=jax.ShapeDtypeStruct((BATCH // 128, 2, 128), jnp.float32),
        mesh=mesh,
        compiler_params=pltpu.CompilerParams(
            needs_layout_passes=False, use_tc_tiling_on_sc=False,
        ),
        scratch_types=[
            pltpu.VMEM((B_PER_W,), jnp.int32),
            pltpu.VMEM((TABLE_WORDS,), jnp.float32),
            pltpu.VMEM((NPAIR_PAD,), jnp.float32),
            pltpu.VMEM((LANES,), jnp.float32),
            pltpu.VMEM((B_PER_W // 128, 2, 128), jnp.float32),
            pltpu.VMEM_SHARED((NPAIR_PAD,), jnp.float32),
            pltpu.SemaphoreType.DMA,
            pltpu.SemaphoreType.DMA,
        ],
    )(
        stimulus_set[:, 0]
        | (stimulus_set[:, 1] << 5)
        | (stimulus_set[:, 2] << 10),
        embedding.T.reshape(-1),
    )
    return out.transpose(0, 2, 1).reshape(BATCH, 2)


# speedup vs baseline: 1.0120x; 1.0120x over previous
import jax
import jax.numpy as jnp
from jax import lax
from jax.experimental import pallas as pl
from jax.experimental.pallas import tpu as pltpu
from jax.experimental.pallas import tpu_sc as plsc

N_STIMULI = 20
N_DIM = 3
BETA = 10.0
GAMMA = 0.001
BATCH = 16384

NUM_CORES = 1
NUM_SUBCORES = 16
LANES = 16
NUM_WORKERS = NUM_CORES * NUM_SUBCORES
B_PER_W = BATCH // NUM_WORKERS
STEPS = B_PER_W // LANES
TABLE_WORDS = (N_STIMULI + 1) * N_DIM
NV = N_STIMULI + 1
NPAIR = NV * NV
NPAIR_PAD = 448
NCHUNK = NPAIR_PAD // LANES


def _sqrt16(x):
    i = plsc.bitcast(x, jnp.int32)
    i = jnp.int32(0x5F3759DF) - lax.shift_right_arithmetic(i, 1)
    y = plsc.bitcast(i, jnp.float32)
    xh = x * jnp.float32(0.5)
    for _ in range(3):
        y = y * (jnp.float32(1.5) - xh * y * y)
    return x * y


def _pair_similarity(emb_v, p):
    q = p // jnp.int32(NV)
    r = p - q * jnp.int32(NV)
    dsq = jnp.full((LANES,), 0.0, jnp.float32)
    for d in range(N_DIM):
        dd = jnp.full((LANES,), d * NV, jnp.int32)
        t = plsc.load_gather(emb_v, [dd + q]) - plsc.load_gather(emb_v, [dd + r])
        dsq = dsq + t * t
    return jnp.exp(jnp.float32(-BETA) * _sqrt16(dsq)) + jnp.float32(GAMMA)


def _sc_body(widx_hbm, emb_hbm, out_hbm,
             w_v, emb_v, stab_v, sbuf_v, po_v, spmem, sem, sem2):
    sid = lax.axis_index("s")
    wid = sid * NUM_CORES + lax.axis_index("c")
    base = wid * B_PER_W

    ce = pltpu.async_copy(emb_hbm, emb_v, sem2)
    cw = pltpu.async_copy(widx_hbm.at[pl.ds(base, B_PER_W)], w_v, sem)
    ce.wait()

    lanes = lax.iota(jnp.int32, LANES)

    p0 = jnp.minimum(sid * LANES + lanes, jnp.int32(NPAIR - 1))
    sbuf_v[...] = _pair_similarity(emb_v, p0)
    pltpu.sync_copy(sbuf_v, spmem.at[pl.ds(sid * LANES, LANES)])

    @pl.when(sid + 16 < NCHUNK)
    def _():
        p1 = jnp.minimum((sid + 16) * LANES + lanes, jnp.int32(NPAIR - 1))
        sbuf_v[...] = _pair_similarity(emb_v, p1)
        pltpu.sync_copy(sbuf_v, spmem.at[pl.ds((sid + 16) * LANES, LANES)])

    plsc.subcore_barrier()
    pltpu.sync_copy(spmem, stab_v)
    cw.wait()

    mask = jnp.full((LANES,), 31, jnp.int32)

    outs = []
    for step in range(STEPS):
        off = step * LANES
        t = off // 128
        c = off % 128
        w = w_v[pl.ds(off, LANES)]
        q21 = (w & mask) * jnp.int32(NV)
        r1 = lax.shift_right_logical(w, 5) & mask
        r2 = lax.shift_right_logical(w, 10)
        s1 = plsc.load_gather(stab_v, [q21 + r1])
        s2 = plsc.load_gather(stab_v, [q21 + r2])
        inv = jnp.float32(1.0) / (s1 + s2)
        po_v[t, 0, pl.ds(c, LANES)] = s1 * inv
        po_v[t, 1, pl.ds(c, LANES)] = s2 * inv
        if c + LANES == 128:
            outs.append(pltpu.async_copy(
                po_v.at[pl.ds(t, 1)],
                out_hbm.at[pl.ds(wid * (B_PER_W // 128) + t, 1)], sem))
    for o in outs:
        o.wait()


@jax.jit
def kernel(stimulus_set, embedding):
    mesh = plsc.VectorSubcoreMesh(
        core_axis_name="c", subcore_axis_name="s",
        num_cores=NUM_CORES, num_subcores=NUM_SUBCORES,
    )
    out = pl.kernel(
        _sc_body,
        out_type=jax.ShapeDtypeStruct((BATCH // 128, 2, 128), jnp.float32),
        mesh=mesh,
        compiler_params=pltpu.CompilerParams(
            needs_layout_passes=False, use_tc_tiling_on_sc=False,
        ),
        scratch_types=[
            pltpu.VMEM((B_PER_W,), jnp.int32),
            pltpu.VMEM((TABLE_WORDS,), jnp.float32),
            pltpu.VMEM((NPAIR_PAD,), jnp.float32),
            pltpu.VMEM((LANES,), jnp.float32),
            pltpu.VMEM((B_PER_W // 128, 2, 128), jnp.float32),
            pltpu.VMEM_SHARED((NPAIR_PAD,), jnp.float32),
            pltpu.SemaphoreType.DMA,
            pltpu.SemaphoreType.DMA,
        ],
    )(
        stimulus_set[:, 0]
        | (stimulus_set[:, 1] << 5)
        | (stimulus_set[:, 2] << 10),
        embedding.T.reshape(-1),
    )
    return out.transpose(0, 2, 1).reshape(BATCH, 2)


# speedup vs baseline: 1.0224x; 1.0103x over previous
import jax
import jax.numpy as jnp
from jax import lax
from jax.experimental import pallas as pl
from jax.experimental.pallas import tpu as pltpu
from jax.experimental.pallas import tpu_sc as plsc

N_STIMULI = 20
N_DIM = 3
BETA = 10.0
GAMMA = 0.001
BATCH = 16384

NUM_CORES = 1
NUM_SUBCORES = 16
LANES = 16
NUM_WORKERS = NUM_CORES * NUM_SUBCORES
B_PER_W = BATCH // NUM_WORKERS
STEPS = B_PER_W // LANES
TABLE_WORDS = (N_STIMULI + 1) * N_DIM
NV = N_STIMULI + 1
NPAIR = NV * NV
NPAIR_PAD = 448
NCHUNK = NPAIR_PAD // LANES


def _sqrt16(x):
    i = plsc.bitcast(x, jnp.int32)
    i = jnp.int32(0x5F3759DF) - lax.shift_right_arithmetic(i, 1)
    y = plsc.bitcast(i, jnp.float32)
    xh = x * jnp.float32(0.5)
    for _ in range(3):
        y = y * (jnp.float32(1.5) - xh * y * y)
    return x * y


def _pair_similarity(emb_v, p):
    q = p // jnp.int32(NV)
    r = p - q * jnp.int32(NV)
    dsq = jnp.full((LANES,), 0.0, jnp.float32)
    for d in range(N_DIM):
        dd = jnp.full((LANES,), d * NV, jnp.int32)
        t = plsc.load_gather(emb_v, [dd + q]) - plsc.load_gather(emb_v, [dd + r])
        dsq = dsq + t * t
    return jnp.exp(jnp.float32(-BETA) * _sqrt16(dsq)) + jnp.float32(GAMMA)


def _sc_body(widx_hbm, emb_hbm, out_hbm,
             w_v, emb_v, stab_v, sbuf_v, po_v, spmem, sem, sem2):
    sid = lax.axis_index("s")
    wid = sid * NUM_CORES + lax.axis_index("c")
    base = wid * B_PER_W

    ce = pltpu.async_copy(emb_hbm, emb_v, sem2)
    cw = pltpu.async_copy(widx_hbm.at[pl.ds(base, B_PER_W)], w_v, sem)
    ce.wait()

    lanes = lax.iota(jnp.int32, LANES)

    p0 = jnp.minimum(sid * LANES + lanes, jnp.int32(NPAIR - 1))
    sbuf_v[...] = _pair_similarity(emb_v, p0)
    pltpu.sync_copy(sbuf_v, spmem.at[pl.ds(sid * LANES, LANES)])

    @pl.when(sid + 16 < NCHUNK)
    def _():
        p1 = jnp.minimum((sid + 16) * LANES + lanes, jnp.int32(NPAIR - 1))
        sbuf_v[...] = _pair_similarity(emb_v, p1)
        pltpu.sync_copy(sbuf_v, spmem.at[pl.ds((sid + 16) * LANES, LANES)])

    plsc.subcore_barrier()
    pltpu.sync_copy(spmem, stab_v)
    cw.wait()

    mask = jnp.full((LANES,), 31, jnp.int32)

    outs = []
    for step in range(STEPS):
        off = step * LANES
        t = off // 128
        c = off % 128
        w = w_v[pl.ds(off, LANES)]
        q21 = (w & mask) * jnp.int32(NV)
        r1 = lax.shift_right_logical(w, 5) & mask
        r2 = lax.shift_right_logical(w, 10)
        s1 = plsc.load_gather(stab_v, [q21 + r1])
        s2 = plsc.load_gather(stab_v, [q21 + r2])
        inv = jnp.float32(1.0) / (s1 + s2)
        po_v[t, 0, pl.ds(c, LANES)] = s1 * inv
        po_v[t, 1, pl.ds(c, LANES)] = s2 * inv
        if c + LANES == 128:
            outs.append(pltpu.async_copy(
                po_v.at[pl.ds(t, 1)],
                out_hbm.at[pl.ds(wid * (B_PER_W // 128) + t, 1)], sem))
    for o in outs:
        o.wait()


@jax.jit
def kernel(stimulus_set, embedding):
    mesh = plsc.VectorSubcoreMesh(
        core_axis_name="c", subcore_axis_name="s",
        num_cores=NUM_CORES, num_subcores=NUM_SUBCORES,
    )
    out = pl.kernel(
        _sc_body,
        out_type=jax.ShapeDtypeStruct((BATCH // 128, 2, 128), jnp.float32),
        mesh=mesh,
        compiler_params=pltpu.CompilerParams(
            needs_layout_passes=False, use_tc_tiling_on_sc=False,
        ),
        scratch_types=[
            pltpu.VMEM((B_PER_W,), jnp.int32),
            pltpu.VMEM((TABLE_WORDS,), jnp.float32),
            pltpu.VMEM((NPAIR_PAD,), jnp.float32),
            pltpu.VMEM((LANES,), jnp.float32),
            pltpu.VMEM((B_PER_W // 128, 2, 128), jnp.float32),
            pltpu.VMEM_SHARED((NPAIR_PAD,), jnp.float32),
            pltpu.SemaphoreType.DMA,
            pltpu.SemaphoreType.DMA,
        ],
    )(
        (stimulus_set * jnp.array([1, 32, 1024], jnp.int32)).sum(axis=1),
        embedding.T.reshape(-1),
    )
    return out.transpose(0, 2, 1).reshape(BATCH, 2)
